# Initial kernel scaffold; baseline (speedup 1.0000x reference)
import jax
import jax.numpy as jnp
from jax.experimental import pallas as pl


def kernel(x_in, pos_in, batch_in, pos_out, batch_out, in_index, out_index, W1, W2, W3, b3):
    raise NotImplementedError("write your pallas kernel here")


# speedup vs baseline: 1.8378x; 1.8378x over previous
import functools

import jax
import jax.numpy as jnp
from jax import lax
from jax.experimental import pallas as pl
from jax.experimental.pallas import tpu as pltpu, tpu_sc as plsc

N = 50000
E = 800000
C_IN = 16
C_MID = 64
C_OUT = 64
MAXN = 64

ROW = 32
NROWS = N * MAXN + 8
TRASH = N * MAXN

NC, NS = 2, 16
NW = NC * NS
EPW = E // NW
CB = 128
NFULL = EPW // CB
TAIL = EPW - NFULL * CB

NB = 200
NBLK = N // NB


def _sc_gather_scatter(table, iidx, slot, dense,
                       idx_v, slot_v, rows_v, idx_t, slot_t, rows_t, sem):
    wid = lax.axis_index("s") * NC + lax.axis_index("c")
    base = wid * EPW

    def body(i, carry):
        off = base + i * CB
        pltpu.sync_copy(iidx.at[pl.ds(off, CB)], idx_v)
        pltpu.sync_copy(slot.at[pl.ds(off, CB)], slot_v)
        pltpu.async_copy(table.at[idx_v], rows_v, sem).wait()
        pltpu.async_copy(rows_v, dense.at[slot_v], sem).wait()
        return carry

    lax.fori_loop(0, NFULL, body, 0)

    offt = base + NFULL * CB
    pltpu.sync_copy(iidx.at[pl.ds(offt, TAIL)], idx_t)
    pltpu.sync_copy(slot.at[pl.ds(offt, TAIL)], slot_t)
    pltpu.async_copy(table.at[idx_t], rows_t, sem).wait()
    pltpu.async_copy(rows_t, dense.at[slot_t], sem).wait()


_sc_call = functools.partial(
    pl.kernel,
    mesh=plsc.VectorSubcoreMesh(core_axis_name="c", subcore_axis_name="s"),
    compiler_params=pltpu.CompilerParams(use_tc_tiling_on_sc=False),
    out_type=jax.ShapeDtypeStruct((NROWS, ROW), jnp.float32),
    scratch_types=[
        pltpu.VMEM((CB,), jnp.int32),
        pltpu.VMEM((CB,), jnp.int32),
        pltpu.VMEM((CB, ROW), jnp.float32),
        pltpu.VMEM((TAIL,), jnp.int32),
        pltpu.VMEM((TAIL,), jnp.int32),
        pltpu.VMEM((TAIL, ROW), jnp.float32),
        pltpu.SemaphoreType.DMA,
    ],
)(_sc_gather_scatter)


def _celu(x):
    return jnp.where(x > 0, x, jnp.exp(jnp.minimum(x, 0.0)) - 1.0)


def _tc_body(dense_ref, por_ref, icol_ref, w1_ref, w2_ref, w3_ref,
             b3_ref, out_ref):
    d = dense_ref[...]
    icol = icol_ref[...]
    valid = icol > 0.0

    pos_loc = d[:, 0:3] - por_ref[...]
    h = _celu(jnp.dot(pos_loc, w1_ref[...], preferred_element_type=jnp.float32))
    h = _celu(jnp.dot(h, w2_ref[...], preferred_element_type=jnp.float32))
    m = jnp.where(valid, h, 0.0)
    xg = jnp.where(valid, d[:, 3:3 + C_IN] * icol, 0.0)

    out = jnp.zeros((NB, C_OUT), jnp.float32)
    for c in range(C_IN):
        z = (m * xg[:, c:c + 1]).reshape(NB, MAXN, C_MID)
        out = out + jnp.dot(jnp.sum(z, axis=1), w3_ref[c],
                            preferred_element_type=jnp.float32)
    out_ref[...] = out + b3_ref[...]


def kernel(x_in, pos_in, batch_in, pos_out, batch_out, in_index, out_index,
           W1, W2, W3, b3):
    starts = jnp.searchsorted(out_index, jnp.arange(N, dtype=jnp.int32),
                              side="left").astype(jnp.int32)
    ends = jnp.searchsorted(out_index, jnp.arange(N, dtype=jnp.int32),
                            side="right").astype(jnp.int32)
    counts = (ends - starts).astype(jnp.float32)
    pos = jnp.arange(E, dtype=jnp.int32) - starts[out_index]
    slot = jnp.where(pos < MAXN, out_index * MAXN + pos, TRASH).astype(jnp.int32)

    table = jnp.concatenate(
        [pos_in, x_in, jnp.zeros((N, ROW - 3 - C_IN), jnp.float32)], axis=1)

    dense = _sc_call(table, in_index, slot)

    inv = jnp.where(counts > 0, 1.0 / jnp.maximum(counts, 1.0), 0.0)
    slot_valid = (jnp.arange(MAXN, dtype=jnp.float32)[None, :]
                  < counts[:, None])
    icol = (slot_valid * inv[:, None]).reshape(N * MAXN, 1)
    por = jnp.broadcast_to(pos_out[:, None, :], (N, MAXN, 3)).reshape(
        N * MAXN, 3)

    out = pl.pallas_call(
        _tc_body,
        grid=(NBLK,),
        in_specs=[
            pl.BlockSpec((NB * MAXN, ROW), lambda i: (i, 0)),
            pl.BlockSpec((NB * MAXN, 3), lambda i: (i, 0)),
            pl.BlockSpec((NB * MAXN, 1), lambda i: (i, 0)),
            pl.BlockSpec((3, C_IN), lambda i: (0, 0)),
            pl.BlockSpec((C_IN, C_MID), lambda i: (0, 0)),
            pl.BlockSpec((C_IN, C_MID, C_OUT), lambda i: (0, 0, 0)),
            pl.BlockSpec((1, C_OUT), lambda i: (0, 0)),
        ],
        out_specs=pl.BlockSpec((NB, C_OUT), lambda i: (i, 0)),
        out_shape=jax.ShapeDtypeStruct((N, C_OUT), jnp.float32),
    )(dense, por, icol, W1, W2,
      W3.reshape(C_IN, C_MID, C_OUT), b3.reshape(1, C_OUT))
    return out


# speedup vs baseline: 1.8547x; 1.0092x over previous
import functools

import jax
import jax.numpy as jnp
from jax import lax
from jax.experimental import pallas as pl
from jax.experimental.pallas import tpu as pltpu, tpu_sc as plsc

N = 50000
E = 800000
C_IN = 16
C_MID = 64
C_OUT = 64
MAXN = 64

ROW = 32
NROWS = N * MAXN + 8
TRASH = N * MAXN

NC, NS = 2, 16
NW = NC * NS
CB = 128
KDEPTH = 4
CPW = 196
EPAD = NW * CPW * CB
QUADS = CPW // KDEPTH

NB = 200
NBLK = N // NB


def _sc_gather_scatter(iidx2d, slot2d, table, dense,
                       idx_v, slot_v, rows_v, sem_g, sem_s):
    wid = lax.axis_index("s") * NC + lax.axis_index("c")
    base = wid * CPW

    def body(i, carry):
        row0 = base + i * KDEPTH
        pltpu.sync_copy(iidx2d.at[pl.ds(row0, KDEPTH)], idx_v)
        pltpu.sync_copy(slot2d.at[pl.ds(row0, KDEPTH)], slot_v)
        gathers = [
            pltpu.async_copy(table.at[idx_v.at[b]], rows_v.at[b], sem_g)
            for b in range(KDEPTH)
        ]
        for d in gathers:
            d.wait()
        scatters = [
            pltpu.async_copy(rows_v.at[b], dense.at[slot_v.at[b]], sem_s)
            for b in range(KDEPTH)
        ]
        for d in scatters:
            d.wait()
        return carry

    lax.fori_loop(0, QUADS, body, 0)


_sc_call = functools.partial(
    pl.kernel,
    mesh=plsc.VectorSubcoreMesh(core_axis_name="c", subcore_axis_name="s"),
    compiler_params=pltpu.CompilerParams(use_tc_tiling_on_sc=False),
    out_type=jax.ShapeDtypeStruct((NROWS, ROW), jnp.float32),
    scratch_types=[
        pltpu.VMEM((KDEPTH, CB), jnp.int32),
        pltpu.VMEM((KDEPTH, CB), jnp.int32),
        pltpu.VMEM((KDEPTH, CB, ROW), jnp.float32),
        pltpu.SemaphoreType.DMA,
        pltpu.SemaphoreType.DMA,
    ],
)(_sc_gather_scatter)


def _celu(x):
    return jnp.where(x > 0, x, jnp.exp(jnp.minimum(x, 0.0)) - 1.0)


def _tc_body(dense_ref, por_ref, icol_ref, w1_ref, w2_ref, w3_ref,
             b3_ref, out_ref):
    d = dense_ref[...]
    icol = icol_ref[...]
    valid = icol > 0.0

    pos_loc = d[:, 0:3] - por_ref[...]
    h = _celu(jnp.dot(pos_loc, w1_ref[...], preferred_element_type=jnp.float32))
    h = _celu(jnp.dot(h, w2_ref[...], preferred_element_type=jnp.float32))
    m = jnp.where(valid, h, 0.0)
    xg = jnp.where(valid, d[:, 3:3 + C_IN] * icol, 0.0)

    out = jnp.zeros((NB, C_OUT), jnp.float32)
    for c in range(C_IN):
        z = (m * xg[:, c:c + 1]).reshape(NB, MAXN, C_MID)
        out = out + jnp.dot(jnp.sum(z, axis=1), w3_ref[c],
                            preferred_element_type=jnp.float32)
    out_ref[...] = out + b3_ref[...]


def kernel(x_in, pos_in, batch_in, pos_out, batch_out, in_index, out_index,
           W1, W2, W3, b3):
    starts = jnp.searchsorted(out_index, jnp.arange(N, dtype=jnp.int32),
                              side="left").astype(jnp.int32)
    ends = jnp.searchsorted(out_index, jnp.arange(N, dtype=jnp.int32),
                            side="right").astype(jnp.int32)
    counts = (ends - starts).astype(jnp.float32)
    pos = jnp.arange(E, dtype=jnp.int32) - starts[out_index]
    slot = jnp.where(pos < MAXN, out_index * MAXN + pos, TRASH).astype(jnp.int32)

    table = jnp.concatenate(
        [pos_in, x_in, jnp.zeros((N, ROW - 3 - C_IN), jnp.float32)], axis=1)

    pad = jnp.zeros((EPAD - E,), jnp.int32)
    iidx2d = jnp.concatenate([in_index, pad]).reshape(EPAD // CB, CB)
    slot2d = jnp.concatenate([slot, pad + TRASH]).reshape(EPAD // CB, CB)

    dense = _sc_call(iidx2d, slot2d, table)

    inv = jnp.where(counts > 0, 1.0 / jnp.maximum(counts, 1.0), 0.0)
    slot_valid = (jnp.arange(MAXN, dtype=jnp.float32)[None, :]
                  < counts[:, None])
    icol = (slot_valid * inv[:, None]).reshape(N * MAXN, 1)
    por = jnp.broadcast_to(pos_out[:, None, :], (N, MAXN, 3)).reshape(
        N * MAXN, 3)

    out = pl.pallas_call(
        _tc_body,
        grid=(NBLK,),
        in_specs=[
            pl.BlockSpec((NB * MAXN, ROW), lambda i: (i, 0)),
            pl.BlockSpec((NB * MAXN, 3), lambda i: (i, 0)),
            pl.BlockSpec((NB * MAXN, 1), lambda i: (i, 0)),
            pl.BlockSpec((3, C_IN), lambda i: (0, 0)),
            pl.BlockSpec((C_IN, C_MID), lambda i: (0, 0)),
            pl.BlockSpec((C_IN, C_MID, C_OUT), lambda i: (0, 0, 0)),
            pl.BlockSpec((1, C_OUT), lambda i: (0, 0)),
        ],
        out_specs=pl.BlockSpec((NB, C_OUT), lambda i: (i, 0)),
        out_shape=jax.ShapeDtypeStruct((N, C_OUT), jnp.float32),
    )(dense, por, icol, W1, W2,
      W3.reshape(C_IN, C_MID, C_OUT), b3.reshape(1, C_OUT))
    return out
